# baseline (device time: 32913 ns/iter reference)
import os

import numpy as np
import jax
import jax.numpy as jnp
from jax import lax
from jax.experimental import pallas as pl
from jax.experimental.pallas import tpu as pltpu

N_DEV = 4
_ABL = int(os.environ.get("ABL", "0"))
B, SQ, D = 2, 256, 768
H_LOC, DH = 4, 64
HD = H_LOC * DH
ROWS = B * SQ


def _rope_consts():
    inv = 1.0 / (10000.0 ** (np.arange(0, DH, 2) / DH))
    pos = np.arange(SQ)[:, None] * inv[None, :]
    cos = np.repeat(np.cos(pos), 2, axis=-1)
    sin = np.repeat(np.sin(pos), 2, axis=-1)
    cos_t = np.tile(cos, (B, H_LOC))
    sin_t = np.tile(sin, (B, H_LOC))
    R = np.zeros((DH, DH), dtype=np.float64)
    for k in range(DH // 2):
        R[2 * k + 1, 2 * k] = -1.0
        R[2 * k, 2 * k + 1] = 1.0
    Rbd = np.kron(np.eye(H_LOC), R)
    return cos_t, sin_t, Rbd


def kernel(x, Wq, Wk, Wv, Wo):
    cos_t, sin_t, Rbd = _rope_consts()
    bf = jnp.bfloat16
    cos_q = jnp.asarray(cos_t * 0.125, dtype=bf)
    sin_q = jnp.asarray(sin_t * 0.125, dtype=bf)
    cos_k = jnp.asarray(cos_t, dtype=bf)
    sin_k = jnp.asarray(sin_t, dtype=bf)
    R_c = jnp.asarray(Rbd, dtype=bf)

    def body(x_ref, wq_ref, wk_ref, wv_ref, wo_ref,
             cosq_ref, sinq_ref, cosk_ref, sink_ref, r_ref,
             out_ref, ctx_ref, wo_src, ctx_full, wo_full,
             wo_ssems, wo_rsems, ctx_ssems, ctx_rsems, loc_sems):
        my = lax.axis_index("i")

        f32 = jnp.float32
        bf16 = jnp.bfloat16

        barrier_sem = pltpu.get_barrier_semaphore()
        for d in range(1, N_DEV):
            pl.semaphore_signal(
                barrier_sem, inc=1,
                device_id=(lax.rem(my + d, N_DEV),),
                device_id_type=pl.DeviceIdType.MESH,
            )
        pl.semaphore_wait(barrier_sem, N_DEV - 1)

        sends = []
        local_copies = []

        wo_src[...] = wo_ref[...].astype(bf16)
        loc = pltpu.make_async_copy(
            wo_src, wo_full.at[pl.ds(my * HD, HD), :], loc_sems.at[0]
        )
        loc.start()
        local_copies.append(loc)
        for d in range(1, N_DEV) if not _ABL else ():
            t = lax.rem(my + d, N_DEV)
            rdma = pltpu.make_async_remote_copy(
                src_ref=wo_src,
                dst_ref=wo_full.at[pl.ds(my * HD, HD), :],
                send_sem=wo_ssems.at[d - 1],
                recv_sem=wo_rsems.at[my],
                device_id=(t,),
                device_id_type=pl.DeviceIdType.MESH,
            )
            rdma.start()
            sends.append(rdma)

        cos_q_v = cosq_ref[...]
        sin_q_v = sinq_ref[...]
        cos_k_v = cosk_ref[...]
        sin_k_v = sink_ref[...]
        R = r_ref[...]
        wq = wq_ref[...].astype(bf16)
        wk = wk_ref[...].astype(bf16)
        wv = wv_ref[...].astype(bf16)

        xf = x_ref[...].reshape(ROWS, D).astype(bf16)
        q = jnp.dot(xf, wq, preferred_element_type=f32).astype(bf16)
        k = jnp.dot(xf, wk, preferred_element_type=f32).astype(bf16)
        v = jnp.dot(xf, wv, preferred_element_type=f32).astype(bf16)
        q_rot = q * cos_q_v + jnp.dot(
            q, R, preferred_element_type=f32).astype(bf16) * sin_q_v
        k_rot = k * cos_k_v + jnp.dot(
            k, R, preferred_element_type=f32).astype(bf16) * sin_k_v

        for b in range(B):
            rows = slice(b * SQ, (b + 1) * SQ)
            for h in range(H_LOC):
                sl = slice(h * DH, (h + 1) * DH)
                qh = q_rot[rows, sl]
                kh = k_rot[rows, sl]
                s = lax.dot_general(
                    qh, kh, (((1,), (1,)), ((), ())),
                    preferred_element_type=f32,
                )
                es = jnp.exp(s)
                recip = 1.0 / jnp.sum(es, axis=-1, keepdims=True)
                ctx = jnp.dot(
                    es.astype(bf16), v[rows, sl], preferred_element_type=f32
                ) * recip
                ctx_ref[b, :, sl] = ctx.astype(bf16)
            loc = pltpu.make_async_copy(
                ctx_ref.at[b],
                ctx_full.at[pl.ds(b * SQ, SQ), pl.ds(my * HD, HD)],
                loc_sems.at[1 + b],
            )
            loc.start()
            local_copies.append(loc)
            for d in range(1, N_DEV) if not _ABL else ():
                t = lax.rem(my + d, N_DEV)
                rdma = pltpu.make_async_remote_copy(
                    src_ref=ctx_ref.at[b],
                    dst_ref=ctx_full.at[pl.ds(b * SQ, SQ), pl.ds(my * HD, HD)],
                    send_sem=ctx_ssems.at[(d - 1) * B + b],
                    recv_sem=ctx_rsems.at[my, b],
                    device_id=(t,),
                    device_id_type=pl.DeviceIdType.MESH,
                )
                rdma.start()
                sends.append(rdma)

        for d in range(1, N_DEV) if not _ABL else ():
            m = lax.rem(my + d, N_DEV)
            pltpu.make_async_remote_copy(
                src_ref=wo_src,
                dst_ref=wo_full.at[pl.ds(m * HD, HD), :],
                send_sem=wo_ssems.at[d - 1],
                recv_sem=wo_rsems.at[m],
                device_id=(my,),
                device_id_type=pl.DeviceIdType.MESH,
            ).wait_recv()
            for b in range(B):
                pltpu.make_async_remote_copy(
                    src_ref=ctx_ref.at[b],
                    dst_ref=ctx_full.at[pl.ds(b * SQ, SQ), pl.ds(m * HD, HD)],
                    send_sem=ctx_ssems.at[(d - 1) * B + b],
                    recv_sem=ctx_rsems.at[m, b],
                    device_id=(my,),
                    device_id_type=pl.DeviceIdType.MESH,
                ).wait_recv()
        for loc in local_copies:
            loc.wait()

        of = jnp.dot(ctx_full[...], wo_full[...], preferred_element_type=f32)
        for b in range(B):
            out_ref[b] = of[b * SQ:(b + 1) * SQ, :]

        for rdma in sends:
            rdma.wait_send()

    return pl.pallas_call(
        body,
        out_shape=jax.ShapeDtypeStruct((B, SQ, D), jnp.float32),
        in_specs=[pl.BlockSpec(memory_space=pltpu.VMEM)] * 10,
        out_specs=pl.BlockSpec(memory_space=pltpu.VMEM),
        scratch_shapes=[
            pltpu.VMEM((B, SQ, HD), jnp.bfloat16),
            pltpu.VMEM((HD, D), jnp.bfloat16),
            pltpu.VMEM((ROWS, N_DEV * HD), jnp.bfloat16),
            pltpu.VMEM((N_DEV * HD, D), jnp.bfloat16),
            pltpu.SemaphoreType.DMA((N_DEV - 1,)),
            pltpu.SemaphoreType.DMA((N_DEV,)),
            pltpu.SemaphoreType.DMA(((N_DEV - 1) * B,)),
            pltpu.SemaphoreType.DMA((N_DEV, B)),
            pltpu.SemaphoreType.DMA((1 + B,)),
        ],
        compiler_params=pltpu.CompilerParams(collective_id=0),
    )(x, Wq, Wk, Wv, Wo, cos_q, sin_q, cos_k, sin_k, R_c)


# device time: 31320 ns/iter; 1.0509x vs baseline; 1.0509x over previous
import os

import numpy as np
import jax
import jax.numpy as jnp
from jax import lax
from jax.experimental import pallas as pl
from jax.experimental.pallas import tpu as pltpu

N_DEV = 4
_ABL = int(os.environ.get("ABL", "0"))
B, SQ, D = 2, 256, 768
H_LOC, DH = 4, 64
HD = H_LOC * DH
ROWS = B * SQ
CH = ROWS // N_DEV


def _rope_consts():
    inv = 1.0 / (10000.0 ** (np.arange(0, DH, 2) / DH))
    pos = np.arange(SQ)[:, None] * inv[None, :]
    cos = np.repeat(np.cos(pos), 2, axis=-1)
    sin = np.repeat(np.sin(pos), 2, axis=-1)
    cos_t = np.tile(cos, (B, H_LOC))
    sin_t = np.tile(sin, (B, H_LOC))
    R = np.zeros((DH, DH), dtype=np.float64)
    for k in range(DH // 2):
        R[2 * k + 1, 2 * k] = -1.0
        R[2 * k, 2 * k + 1] = 1.0
    Rbd = np.kron(np.eye(H_LOC), R)
    return cos_t, sin_t, Rbd


def kernel(x, Wq, Wk, Wv, Wo):
    cos_t, sin_t, Rbd = _rope_consts()
    bf = jnp.bfloat16
    cos_q = jnp.asarray(cos_t * 0.125, dtype=bf)
    sin_q = jnp.asarray(sin_t * 0.125, dtype=bf)
    cos_k = jnp.asarray(cos_t, dtype=bf)
    sin_k = jnp.asarray(sin_t, dtype=bf)
    R_c = jnp.asarray(Rbd, dtype=bf)

    def body(x_ref, wq_ref, wk_ref, wv_ref, wo_ref,
             cosq_ref, sinq_ref, cosk_ref, sink_ref, r_ref,
             out_ref, ctx_ref, sbuf, agsrc, rs_buf, ag_buf,
             rs_ssems, rs_rsems, ag_ssems, ag_rsems):
        my = lax.axis_index("i")

        f32 = jnp.float32
        bf16 = jnp.bfloat16

        barrier_sem = pltpu.get_barrier_semaphore()
        for d in range(1, N_DEV):
            pl.semaphore_signal(
                barrier_sem, inc=1,
                device_id=(lax.rem(my + d, N_DEV),),
                device_id_type=pl.DeviceIdType.MESH,
            )
        pl.semaphore_wait(barrier_sem, N_DEV - 1)

        wq = wq_ref[...].astype(bf16)
        wk = wk_ref[...].astype(bf16)
        wv = wv_ref[...].astype(bf16)
        wo = wo_ref[...].astype(bf16)

        xf = x_ref[...].reshape(ROWS, D).astype(bf16)
        q = jnp.dot(xf, wq, preferred_element_type=f32).astype(bf16)
        k = jnp.dot(xf, wk, preferred_element_type=f32).astype(bf16)
        v = jnp.dot(xf, wv, preferred_element_type=f32).astype(bf16)
        q_rot = q * cosq_ref[...] + jnp.dot(
            q, r_ref[...], preferred_element_type=f32).astype(bf16) * sinq_ref[...]
        k_rot = k * cosk_ref[...] + jnp.dot(
            k, r_ref[...], preferred_element_type=f32).astype(bf16) * sink_ref[...]

        sends = []

        for b in range(B):
            rows = slice(b * SQ, (b + 1) * SQ)
            for h in range(H_LOC):
                sl = slice(h * DH, (h + 1) * DH)
                s = lax.dot_general(
                    q_rot[rows, sl], k_rot[rows, sl],
                    (((1,), (1,)), ((), ())),
                    preferred_element_type=f32,
                )
                es = jnp.exp(s)
                recip = 1.0 / jnp.sum(es, axis=-1, keepdims=True)
                ctx = jnp.dot(
                    es.astype(bf16), v[rows, sl], preferred_element_type=f32
                ) * recip
                ctx_ref[b, :, sl] = ctx.astype(bf16)
            for half in range(2):
                c = 2 * b + half
                pc = jnp.dot(
                    ctx_ref[b, half * CH:(half + 1) * CH, :], wo,
                    preferred_element_type=f32,
                )
                sbuf[c] = pc.astype(bf16)
                if not _ABL:
                    rdma = pltpu.make_async_remote_copy(
                        src_ref=sbuf.at[c],
                        dst_ref=rs_buf.at[my],
                        send_sem=rs_ssems.at[c],
                        recv_sem=rs_rsems.at[my],
                        device_id=(c,),
                        device_id_type=pl.DeviceIdType.MESH,
                    )
                    rdma.start()
                    sends.append(rdma)

        for m in range(N_DEV) if not _ABL else ():
            pltpu.make_async_remote_copy(
                src_ref=rs_buf.at[m],
                dst_ref=rs_buf.at[m],
                send_sem=rs_ssems.at[m],
                recv_sem=rs_rsems.at[m],
                device_id=(my,),
                device_id_type=pl.DeviceIdType.MESH,
            ).wait_recv()
        red = rs_buf[0].astype(f32)
        for m in range(1, N_DEV):
            red = red + rs_buf[m].astype(f32)
        agsrc[...] = red.astype(bf16)

        for t in range(N_DEV) if not _ABL else ():
            rdma = pltpu.make_async_remote_copy(
                src_ref=agsrc,
                dst_ref=ag_buf.at[my],
                send_sem=ag_ssems.at[t],
                recv_sem=ag_rsems.at[my],
                device_id=(t,),
                device_id_type=pl.DeviceIdType.MESH,
            )
            rdma.start()
            sends.append(rdma)

        for c in range(N_DEV):
            if not _ABL:
                pltpu.make_async_remote_copy(
                    src_ref=ag_buf.at[c],
                    dst_ref=ag_buf.at[c],
                    send_sem=ag_ssems.at[c],
                    recv_sem=ag_rsems.at[c],
                    device_id=(my,),
                    device_id_type=pl.DeviceIdType.MESH,
                ).wait_recv()
            out_ref[c // 2, (c % 2) * CH:(c % 2) * CH + CH, :] = (
                ag_buf[c].astype(f32)
            )

        for rdma in sends:
            rdma.wait_send()

    return pl.pallas_call(
        body,
        out_shape=jax.ShapeDtypeStruct((B, SQ, D), jnp.float32),
        in_specs=[pl.BlockSpec(memory_space=pltpu.VMEM)] * 10,
        out_specs=pl.BlockSpec(memory_space=pltpu.VMEM),
        scratch_shapes=[
            pltpu.VMEM((B, SQ, HD), jnp.bfloat16),
            pltpu.VMEM((N_DEV, CH, D), jnp.bfloat16),
            pltpu.VMEM((CH, D), jnp.bfloat16),
            pltpu.VMEM((N_DEV, CH, D), jnp.bfloat16),
            pltpu.VMEM((N_DEV, CH, D), jnp.bfloat16),
            pltpu.SemaphoreType.DMA((N_DEV,)),
            pltpu.SemaphoreType.DMA((N_DEV,)),
            pltpu.SemaphoreType.DMA((N_DEV,)),
            pltpu.SemaphoreType.DMA((N_DEV,)),
        ],
        compiler_params=pltpu.CompilerParams(collective_id=0),
    )(x, Wq, Wk, Wv, Wo, cos_q, sin_q, cos_k, sin_k, R_c)
